# baseline (speedup 1.0000x reference)
import jax
import jax.numpy as jnp
from jax import lax
from jax.experimental import pallas as pl
from jax.experimental.pallas import tpu as pltpu

N = 400
B = 4
D = 128
EPS = 1e-5
_F32 = jnp.float32


def _ln(x, g, b):
    mu = jnp.mean(x, axis=-1, keepdims=True)
    var = jnp.mean((x - mu) ** 2, axis=-1, keepdims=True)
    return (x - mu) * lax.rsqrt(var + EPS) * g + b


def _dot(a, c):
    return jnp.dot(a, c, preferred_element_type=_F32)


def _tdot(a, c):
    return lax.dot_general(a, c, (((0,), (0,)), ((), ())),
                           preferred_element_type=_F32)


_SMALL = [(1, D), (D, D), (1, D), (1, D), (1, D),
          (D, 128), (1, 128), (1, 128), (1, 128),
          (128, 64), (1, 64), (1, 64), (1, 64),
          (64, 4), (1, 4)]


def _fused_kernel(*refs):
    sc_ref, W1_ref = refs[0], refs[1]
    hbm = refs[2:17]
    logits_ref, zp_ref = refs[17], refs[18]
    vm = refs[19:34]
    sems = refs[34]

    copies = [pltpu.make_async_copy(h, v, sems.at[i])
              for i, (h, v) in enumerate(zip(hbm, vm))]
    for c in copies:
        c.start()

    SS = sc_ref[...]
    Sb = [SS[i * N:(i + 1) * N, :] for i in range(B)]

    rb = lax.broadcasted_iota(jnp.int32, (B * N, B), 0) // N
    cb = lax.broadcasted_iota(jnp.int32, (B * N, B), 1)
    O = (rb == cb).astype(_F32)
    degs = _tdot(SS, O)
    dinv_all = jnp.where(degs > 0, lax.rsqrt(degs), 0.0)
    dinv = [dinv_all[:, i:i + 1] for i in range(B)]

    h_all = _dot(SS, W1_ref[...])

    copies[0].wait()
    b1 = vm[0][...]
    x1 = []
    for i in range(B):
        h = h_all[i * N:(i + 1) * N, :]
        x1.append(jnp.maximum(
            _tdot(Sb[i], h * dinv[i]) * dinv[i] + b1, 0.0))

    for c in copies[1:5]:
        c.wait()
    W2, b2, lnE_g, lnE_b = (vm[i][...] for i in range(1, 5))
    h2_all = _dot(jnp.concatenate(x1, axis=0), W2)
    for i in range(B):
        h = h2_all[i * N:(i + 1) * N, :]
        y = _tdot(Sb[i], h * dinv[i]) * dinv[i] + b2
        y = _ln(y, lnE_g, lnE_b)
        zp_ref[pl.ds(i, 1), :] = jnp.mean(y, axis=0, keepdims=True)

    for c in copies[5:]:
        c.wait()
    (fc1_W, fc1_b, ln1_g, ln1_b, fc2_W, fc2_b, ln2_g, ln2_b,
     fc3_W, fc3_b) = (vm[i][...] for i in range(5, 15))
    z = zp_ref[...]
    hh = _dot(z, fc1_W) + fc1_b
    hh = jnp.maximum(_ln(hh, ln1_g, ln1_b), 0.0)
    hh = _dot(hh, fc2_W) + fc2_b
    hh = jnp.maximum(_ln(hh, ln2_g, ln2_b), 0.0)
    logits_ref[...] = _dot(hh, fc3_W) + fc3_b


def kernel(sc_matrix, W1, b1, W2, b2, lnE_g, lnE_b, fc1_W, fc1_b, ln1_g,
           ln1_b, fc2_W, fc2_b, ln2_g, ln2_b, fc3_W, fc3_b):
    r2 = lambda v: v.reshape(1, -1)
    hbm_spec = pl.BlockSpec(memory_space=pltpu.MemorySpace.HBM)
    vmem_spec = pl.BlockSpec(memory_space=pltpu.MemorySpace.VMEM)
    logits, zp = pl.pallas_call(
        _fused_kernel,
        in_specs=[vmem_spec, vmem_spec] + [hbm_spec] * 15,
        out_specs=[vmem_spec, vmem_spec],
        out_shape=[
            jax.ShapeDtypeStruct((B, 4), _F32),
            jax.ShapeDtypeStruct((B, D), _F32),
        ],
        scratch_shapes=[pltpu.VMEM(s, _F32) for s in _SMALL]
        + [pltpu.SemaphoreType.DMA((15,))],
    )(sc_matrix.reshape(B * N, N), W1, r2(b1), W2, r2(b2), r2(lnE_g),
      r2(lnE_b), fc1_W, r2(fc1_b), r2(ln1_g), r2(ln1_b),
      fc2_W, r2(fc2_b), r2(ln2_g), r2(ln2_b),
      fc3_W, r2(fc3_b))
    return (logits, zp)


# speedup vs baseline: 1.0675x; 1.0675x over previous
import jax
import jax.numpy as jnp
from jax import lax
from jax.experimental import pallas as pl

N = 400
B = 4
D = 128
EPS = 1e-5
_F32 = jnp.float32


def _ln(x, g, b):
    mu = jnp.mean(x, axis=-1, keepdims=True)
    var = jnp.mean((x - mu) ** 2, axis=-1, keepdims=True)
    return (x - mu) * lax.rsqrt(var + EPS) * g + b


def _dot(a, c):
    return jnp.dot(a, c, preferred_element_type=_F32)


def _tdot(a, c):
    return lax.dot_general(a, c, (((0,), (0,)), ((), ())),
                           preferred_element_type=_F32)


def _fused_kernel(sc_ref, W1_ref, b1_ref, W2_ref, b2_ref, lnEg_ref, lnEb_ref,
                  fc1W_ref, fc1b_ref, ln1g_ref, ln1b_ref,
                  fc2W_ref, fc2b_ref, ln2g_ref, ln2b_ref,
                  fc3W_ref, fc3b_ref,
                  logits_ref, zp_ref):
    SS = sc_ref[...]
    Sb = [SS[i * N:(i + 1) * N, :] for i in range(B)]

    rb = lax.broadcasted_iota(jnp.int32, (B * N, B), 0) // N
    cb = lax.broadcasted_iota(jnp.int32, (B * N, B), 1)
    O = (rb == cb).astype(_F32)
    degs = _tdot(SS, O)
    dinv_all = jnp.where(degs > 0, lax.rsqrt(degs), 0.0)
    dinv = [dinv_all[:, i:i + 1] for i in range(B)]

    h_all = _dot(SS, W1_ref[...])
    x1 = []
    for i in range(B):
        h = h_all[i * N:(i + 1) * N, :]
        x1.append(jnp.maximum(
            _tdot(Sb[i], h * dinv[i]) * dinv[i] + b1_ref[...], 0.0))

    h2_all = _dot(jnp.concatenate(x1, axis=0), W2_ref[...])
    for i in range(B):
        h = h2_all[i * N:(i + 1) * N, :]
        y = _tdot(Sb[i], h * dinv[i]) * dinv[i] + b2_ref[...]
        y = _ln(y, lnEg_ref[...], lnEb_ref[...])
        zp_ref[pl.ds(i, 1), :] = jnp.mean(y, axis=0, keepdims=True)

    z = zp_ref[...]
    hh = _dot(z, fc1W_ref[...]) + fc1b_ref[...]
    hh = jnp.maximum(_ln(hh, ln1g_ref[...], ln1b_ref[...]), 0.0)
    hh = _dot(hh, fc2W_ref[...]) + fc2b_ref[...]
    hh = jnp.maximum(_ln(hh, ln2g_ref[...], ln2b_ref[...]), 0.0)
    logits_ref[...] = _dot(hh, fc3W_ref[...]) + fc3b_ref[...]


def kernel(sc_matrix, W1, b1, W2, b2, lnE_g, lnE_b, fc1_W, fc1_b, ln1_g,
           ln1_b, fc2_W, fc2_b, ln2_g, ln2_b, fc3_W, fc3_b):
    r2 = lambda v: v.reshape(1, -1)
    logits, zp = pl.pallas_call(
        _fused_kernel,
        out_shape=[
            jax.ShapeDtypeStruct((B, 4), _F32),
            jax.ShapeDtypeStruct((B, D), _F32),
        ],
    )(sc_matrix.reshape(B * N, N), W1, r2(b1), W2, r2(b2), r2(lnE_g),
      r2(lnE_b), fc1_W, r2(fc1_b), r2(ln1_g), r2(ln1_b),
      fc2_W, r2(fc2_b), r2(ln2_g), r2(ln2_b),
      fc3_W, r2(fc3_b))
    return (logits, zp)
